# Initial kernel scaffold; baseline (speedup 1.0000x reference)
import jax
import jax.numpy as jnp
from jax.experimental import pallas as pl


def kernel(x_in, tables):
    raise NotImplementedError("write your pallas kernel here")


# speedup vs baseline: 1.1238x; 1.1238x over previous
import functools

import jax
import jax.numpy as jnp
from jax import lax
from jax.experimental import pallas as pl
from jax.experimental.pallas import tpu as pltpu
from jax.experimental.pallas import tpu_sc as plsc

N_CONT = 13
N_CAT = 26
VOCAB = 100000
EDIM = 16
BATCH = 16384

_INFO = plsc.get_sparse_core_info()
_NC = _INFO.num_cores
_NS = _INFO.num_subcores
_L = _INFO.num_lanes
_NW = _NC * _NS

_TOTAL_ROWS = BATCH * N_CAT
_PER_W = _TOTAL_ROWS // _NW
_CHUNK = 3328
_NCHUNK = _PER_W // _CHUNK
_VECS = _CHUNK // _L


def _emb_gather(x_cat_flat, table_flat):
  mesh = plsc.VectorSubcoreMesh(core_axis_name="c", subcore_axis_name="s")

  @functools.partial(
      pl.kernel,
      mesh=mesh,
      out_type=jax.ShapeDtypeStruct((_TOTAL_ROWS, EDIM), jnp.float32),
      compiler_params=pltpu.CompilerParams(use_tc_tiling_on_sc=False),
      scratch_types=[
          pltpu.VMEM((_CHUNK,), jnp.float32),
          pltpu.VMEM((_CHUNK,), jnp.int32),
          pltpu.VMEM((_CHUNK, EDIM), jnp.float32),
          pltpu.SemaphoreType.DMA,
      ],
  )
  def k(xcat_hbm, table_hbm, out_hbm, xbuf, idxbuf, rowbuf, sem):
    wid = lax.axis_index("s") * _NC + lax.axis_index("c")
    base_w = wid * _PER_W
    lane = lax.iota(jnp.int32, _L)

    def chunk_body(c, carry):
      base = pl.multiple_of(base_w + c * _CHUNK, 8)
      pltpu.sync_copy(xcat_hbm.at[pl.ds(base, _CHUNK)], xbuf)

      def vec_body(j, carry2):
        off = pl.multiple_of(j * _L, _L)
        v = xbuf[pl.ds(off, _L)].astype(jnp.int32)
        pos = base + off + lane
        idxbuf[pl.ds(off, _L)] = v + lax.rem(pos, N_CAT) * VOCAB
        return carry2

      lax.fori_loop(0, _VECS, vec_body, 0, unroll=4)

      pltpu.async_copy(table_hbm.at[idxbuf], rowbuf, sem).wait()
      pltpu.sync_copy(rowbuf, out_hbm.at[pl.ds(base, _CHUNK)])
      return carry

    lax.fori_loop(0, _NCHUNK, chunk_body, 0)

  return k(x_cat_flat, table_flat)


def kernel(x_in, tables):
  x_cat_flat = x_in[:, N_CONT:].reshape(-1)
  table_flat = tables.reshape(N_CAT * VOCAB, EDIM)
  emb = _emb_gather(x_cat_flat, table_flat)
  x = emb.reshape(BATCH, N_CAT * EDIM)
  return jnp.concatenate([x, x_in[:, :N_CONT]], axis=1)
